# speedup vs baseline: 1.0432x; 1.0432x over previous
import functools

import jax
import jax.numpy as jnp
from jax import lax
from jax.experimental import pallas as pl
from jax.experimental.pallas import tpu as pltpu
from jax.experimental.pallas import tpu_sc as plsc

NC = 2
NS = 16
LANES = 16


def _emb_kernel(N, T, D, n_per, CH, idx_hbm, tok_hbm, pos_hbm, out_hbm,
                idx_v, tokbuf, outbuf, sem):
    wid = lax.axis_index("s") * NC + lax.axis_index("c")
    base = wid * n_per
    t_base = lax.rem(base, T)
    pltpu.sync_copy(idx_hbm.at[pl.ds(base, n_per)], idx_v)
    n_ch = n_per // CH
    vregs_per_row = D // LANES

    def chunk_body(c, _):
        cp = pltpu.async_copy(tok_hbm.at[idx_v.at[pl.ds(c * CH, CH)]],
                              tokbuf, sem)
        pltpu.sync_copy(pos_hbm.at[pl.ds(t_base + c * CH, CH)], outbuf)
        cp.wait()

        def add_row(r, _):
            for j in range(vregs_per_row):
                sl = pl.ds(j * LANES, LANES)
                plsc.addupdate(outbuf.at[r, sl], tokbuf[r, sl])
            return 0

        lax.fori_loop(0, CH, add_row, 0)
        pltpu.sync_copy(outbuf, out_hbm.at[pl.ds(base + c * CH, CH)])
        return 0

    lax.fori_loop(0, n_ch, chunk_body, 0)


def kernel(idx, token_table, pos_table):
    B, T = idx.shape
    V, D = token_table.shape
    N = B * T
    NW = NC * NS
    n_per = N // NW
    CH = 32

    idx_flat = idx.reshape(N).astype(jnp.int32)

    body = functools.partial(_emb_kernel, N, T, D, n_per, CH)
    f = pl.kernel(
        body,
        out_type=jax.ShapeDtypeStruct((N, D), jnp.float32),
        mesh=plsc.VectorSubcoreMesh(core_axis_name="c", subcore_axis_name="s"),
        scratch_types=[
            pltpu.VMEM((n_per,), jnp.int32),
            pltpu.VMEM((CH, D), jnp.float32),
            pltpu.VMEM((CH, D), jnp.float32),
            pltpu.SemaphoreType.DMA,
        ],
    )
    out = f(idx_flat, token_table, pos_table)
    return out.reshape(B, T, D)
